# baseline (speedup 1.0000x reference)
import functools

import jax
import jax.numpy as jnp
from jax import lax
from jax.experimental import pallas as pl
from jax.experimental.pallas import tpu as pltpu
from jax.experimental.pallas import tpu_sc as plsc

NUM_WORKERS = 32
SPLIT = 128
NBUF = 2


def _make_lookup(b, s, d):
  rows_per_w = b // NUM_WORKERS
  n_pairs = rows_per_w // 2
  mesh = plsc.VectorSubcoreMesh(core_axis_name="c", subcore_axis_name="s")

  @functools.partial(
      pl.kernel,
      out_type=jax.ShapeDtypeStruct((b, s, d), jnp.float32),
      mesh=mesh,
      scratch_types=[
          pltpu.VMEM((rows_per_w, s), jnp.int32),
          pltpu.VMEM((NBUF, 2, s, d), jnp.float32),
          pltpu.SemaphoreType.DMA,
          pltpu.SemaphoreType.DMA,
          pltpu.SemaphoreType.DMA,
          pltpu.SemaphoreType.DMA,
      ],
  )
  def lookup(idx_hbm, table_hbm, out_hbm, idx_v, rows_v, g0, g1, w0, w1):
    gsems = (g0, g1)
    wsems = (w0, w1)
    wid = lax.axis_index("s") * 2 + lax.axis_index("c")
    base = wid * rows_per_w
    pltpu.sync_copy(idx_hbm.at[pl.ds(base, rows_per_w)], idx_v)

    def gathers(p, slot):
      for h in range(2):
        pltpu.async_copy(
            table_hbm.at[idx_v.at[2 * p + h, pl.ds(0, SPLIT)]],
            rows_v.at[slot, h, pl.ds(0, SPLIT)],
            gsems[slot],
        )
        pltpu.async_copy(
            table_hbm.at[idx_v.at[2 * p + h, pl.ds(SPLIT, s - SPLIT)]],
            rows_v.at[slot, h, pl.ds(SPLIT, s - SPLIT)],
            gsems[slot],
        )

    def wait_gathers(p, slot):
      for h in range(2):
        pltpu.make_async_copy(
            table_hbm.at[idx_v.at[2 * p + h, pl.ds(0, SPLIT)]],
            rows_v.at[slot, h, pl.ds(0, SPLIT)],
            gsems[slot],
        ).wait()
        pltpu.make_async_copy(
            table_hbm.at[idx_v.at[2 * p + h, pl.ds(SPLIT, s - SPLIT)]],
            rows_v.at[slot, h, pl.ds(SPLIT, s - SPLIT)],
            gsems[slot],
        ).wait()

    def write(p, slot):
      pltpu.async_copy(
          rows_v.at[slot], out_hbm.at[pl.ds(base + 2 * p, 2)], wsems[slot]
      )

    def wait_write(p, slot):
      pltpu.make_async_copy(
          rows_v.at[slot], out_hbm.at[pl.ds(base + 2 * p, 2)], wsems[slot]
      ).wait()

    gathers(0, 0)
    for p in range(n_pairs):
      slot = p % NBUF
      nslot = (p + 1) % NBUF
      if p + 1 < n_pairs:
        if p >= 1:
          wait_write(p - 1, nslot)
        gathers(p + 1, nslot)
      wait_gathers(p, slot)
      write(p, slot)
    wait_write(n_pairs - 2, (n_pairs - 2) % NBUF)
    wait_write(n_pairs - 1, (n_pairs - 1) % NBUF)

  return lookup


def kernel(x, table):
  b, s = x.shape
  v, d = table.shape
  return _make_lookup(b, s, d)(x, table)


# speedup vs baseline: 1.0216x; 1.0216x over previous
import functools

import jax
import jax.numpy as jnp
from jax import lax
from jax.experimental import pallas as pl
from jax.experimental.pallas import tpu as pltpu
from jax.experimental.pallas import tpu_sc as plsc

NUM_WORKERS = 32
SPLIT = 128
NBUF = 2


def _make_lookup(b, s, d):
  rows_per_w = b // NUM_WORKERS
  n_pairs = rows_per_w // 2
  n_iters = n_pairs // 2
  mesh = plsc.VectorSubcoreMesh(core_axis_name="c", subcore_axis_name="s")

  @functools.partial(
      pl.kernel,
      out_type=jax.ShapeDtypeStruct((b, s, d), jnp.float32),
      mesh=mesh,
      scratch_types=[
          pltpu.VMEM((rows_per_w, s), jnp.int32),
          pltpu.VMEM((NBUF, 2, s, d), jnp.float32),
          pltpu.SemaphoreType.DMA,
          pltpu.SemaphoreType.DMA,
          pltpu.SemaphoreType.DMA,
          pltpu.SemaphoreType.DMA,
      ],
  )
  def lookup(idx_hbm, table_hbm, out_hbm, idx_v, rows_v, g0, g1, w0, w1):
    gsems = (g0, g1)
    wsems = (w0, w1)
    wid = lax.axis_index("s") * 2 + lax.axis_index("c")
    base = wid * rows_per_w
    pltpu.sync_copy(idx_hbm.at[pl.ds(base, rows_per_w)], idx_v)

    def gathers(p, slot):
      for h in range(2):
        pltpu.async_copy(
            table_hbm.at[idx_v.at[2 * p + h, pl.ds(0, SPLIT)]],
            rows_v.at[slot, h, pl.ds(0, SPLIT)],
            gsems[slot],
        )
        pltpu.async_copy(
            table_hbm.at[idx_v.at[2 * p + h, pl.ds(SPLIT, s - SPLIT)]],
            rows_v.at[slot, h, pl.ds(SPLIT, s - SPLIT)],
            gsems[slot],
        )

    def wait_gathers(p, slot):
      for h in range(2):
        pltpu.make_async_copy(
            table_hbm.at[idx_v.at[2 * p + h, pl.ds(0, SPLIT)]],
            rows_v.at[slot, h, pl.ds(0, SPLIT)],
            gsems[slot],
        ).wait()
        pltpu.make_async_copy(
            table_hbm.at[idx_v.at[2 * p + h, pl.ds(SPLIT, s - SPLIT)]],
            rows_v.at[slot, h, pl.ds(SPLIT, s - SPLIT)],
            gsems[slot],
        ).wait()

    def write(p, slot):
      pltpu.async_copy(
          rows_v.at[slot], out_hbm.at[pl.ds(base + 2 * p, 2)], wsems[slot]
      )

    def wait_write(p, slot):
      pltpu.make_async_copy(
          rows_v.at[slot], out_hbm.at[pl.ds(base + 2 * p, 2)], wsems[slot]
      ).wait()

    gathers(0, 0)

    def body(q, carry):
      p0 = 2 * q
      p1 = p0 + 1

      @pl.when(q >= 1)
      def _():
        wait_write(p0 - 1, 1)

      gathers(p1, 1)
      wait_gathers(p0, 0)
      write(p0, 0)

      @pl.when(q < n_iters - 1)
      def _():
        wait_write(p0, 0)
        gathers(p1 + 1, 0)

      wait_gathers(p1, 1)
      write(p1, 1)
      return carry

    lax.fori_loop(0, n_iters, body, 0)
    wait_write(n_pairs - 2, 0)
    wait_write(n_pairs - 1, 1)

  return lookup


def kernel(x, table):
  b, s = x.shape
  v, d = table.shape
  return _make_lookup(b, s, d)(x, table)


# speedup vs baseline: 1.0254x; 1.0037x over previous
import functools

import jax
import jax.numpy as jnp
from jax import lax
from jax.experimental import pallas as pl
from jax.experimental.pallas import tpu as pltpu
from jax.experimental.pallas import tpu_sc as plsc

NUM_WORKERS = 32
SPLIT = 128
NBUF = 4
AHEAD = 3


def _make_lookup(b, s, d):
  rows_per_w = b // NUM_WORKERS
  n_iters = rows_per_w // NBUF
  tail = rows_per_w - NBUF * n_iters
  mesh = plsc.VectorSubcoreMesh(core_axis_name="c", subcore_axis_name="s")

  @functools.partial(
      pl.kernel,
      out_type=jax.ShapeDtypeStruct((b, s, d), jnp.float32),
      mesh=mesh,
      scratch_types=[
          pltpu.VMEM((rows_per_w, s), jnp.int32),
          pltpu.VMEM((NBUF, 1, s, d), jnp.float32),
      ]
      + [pltpu.SemaphoreType.DMA] * (2 * NBUF),
  )
  def lookup(idx_hbm, table_hbm, out_hbm, idx_v, rows_v, *sems):
    gsems = sems[:NBUF]
    wsems = sems[NBUF:]
    wid = lax.axis_index("s") * 2 + lax.axis_index("c")
    base = wid * rows_per_w
    pltpu.sync_copy(idx_hbm.at[pl.ds(base, rows_per_w)], idx_v)

    def gathers(r, slot):
      pltpu.async_copy(
          table_hbm.at[idx_v.at[r, pl.ds(0, SPLIT)]],
          rows_v.at[slot, 0, pl.ds(0, SPLIT)],
          gsems[slot],
      )
      pltpu.async_copy(
          table_hbm.at[idx_v.at[r, pl.ds(SPLIT, s - SPLIT)]],
          rows_v.at[slot, 0, pl.ds(SPLIT, s - SPLIT)],
          gsems[slot],
      )

    def wait_gathers(r, slot):
      pltpu.make_async_copy(
          table_hbm.at[idx_v.at[r, pl.ds(0, SPLIT)]],
          rows_v.at[slot, 0, pl.ds(0, SPLIT)],
          gsems[slot],
      ).wait()
      pltpu.make_async_copy(
          table_hbm.at[idx_v.at[r, pl.ds(SPLIT, s - SPLIT)]],
          rows_v.at[slot, 0, pl.ds(SPLIT, s - SPLIT)],
          gsems[slot],
      ).wait()

    def write(r, slot):
      pltpu.async_copy(
          rows_v.at[slot], out_hbm.at[pl.ds(base + r, 1)], wsems[slot]
      )

    def wait_write(r, slot):
      pltpu.make_async_copy(
          rows_v.at[slot], out_hbm.at[pl.ds(base + r, 1)], wsems[slot]
      ).wait()

    for r in range(AHEAD):
      gathers(r, r)

    def step(r, islot):
      g = r + AHEAD
      gslot = (islot + AHEAD) % NBUF

      @pl.when(jnp.logical_and(g >= NBUF, g < rows_per_w))
      def _():
        wait_write(g - NBUF, gslot)

      @pl.when(g < rows_per_w)
      def _():
        gathers(g, gslot)

      wait_gathers(r, islot)
      write(r, islot)

    def body(q, carry):
      r0 = NBUF * q
      for i in range(NBUF):
        step(r0 + i, i)
      return carry

    lax.fori_loop(0, n_iters, body, 0)
    for i in range(tail):
      step(NBUF * n_iters + i, i)
    for i in range(NBUF):
      r_last = rows_per_w - NBUF + i
      wait_write(r_last, (r_last) % NBUF)

  return lookup


def kernel(x, table):
  b, s = x.shape
  v, d = table.shape
  return _make_lookup(b, s, d)(x, table)
